# baseline (speedup 1.0000x reference)
import functools

import jax
import jax.numpy as jnp
from jax import lax
from jax.experimental import pallas as pl
from jax.experimental.pallas import tpu as pltpu
from jax.experimental.pallas import tpu_sc as plsc

_B = 4096
_C = 26
_D = 1024
_K = 32
_L = 16
_NC = 2
_NS = 16
_NW = _NC * _NS
_BPW = _B // _NW
_G = 16
_NG = _BPW // _G


def _rev(x):
    return lax.rev(x, (0,))


def _sort16(x):
    return plsc.sort_key_val(x, x)[0]


def _sort16d(x):
    return plsc.sort_key_val(x, x, descending=True)[0]


def _merge_pair(v0, v1, R0, R1):
    a = _sort16(v0)
    bd = _sort16d(v1)
    u0d = _sort16d(jnp.minimum(a, bd))
    u1d = _sort16d(jnp.maximum(a, bd))
    m0 = jnp.minimum(R0, u1d)
    m1 = jnp.minimum(R1, u0d)
    lo = jnp.minimum(m0, m1)
    hi = jnp.maximum(m0, m1)
    return _sort16(lo), _sort16(hi)


def _sc_body(encoded, labels, out, lab_v, bufs, out_v, sem_a, sem_b, sem_c):
    wid = lax.axis_index("s") * _NC + lax.axis_index("c")
    base = wid * _BPW

    pltpu.sync_copy(labels.at[pl.ds(base, _BPW)], lab_v)
    lane = lax.iota(jnp.int32, _L)
    lane_hi = lane + jnp.int32(_L)

    def start(g, m, sem):
        for j in range(_G // _L):
            chunk = lab_v[pl.ds(g * _G + j * _L, _L)]
            idx = chunk * jnp.int32(_B) + (base + g * _G + j * _L + lane)
            pltpu.async_copy(
                encoded.at[idx], bufs.at[m, pl.ds(j * _L, _L)], sem)

    def drain(m, sem):
        pltpu.make_async_copy(encoded.at[pl.ds(0, _G)], bufs.at[m], sem).wait()

    def process(m, g):
        inf16 = jnp.full((_L,), jnp.inf, jnp.float32)

        def finish(R, r):
            col = jnp.full((_L,), g * _G + r, jnp.int32)
            plsc.store_scatter(out_v, [lane, col], R[0])
            plsc.store_scatter(out_v, [lane_hi, col], R[1])

        def row_body(r, carry):
            rows = [16 * r + k for k in range(16)]

            def chunk_body(c, R):
                off = c * 64

                def duo(row, Rq):
                    q0, q1 = _merge_pair(
                        bufs[m, row, pl.ds(off, _L)],
                        bufs[m, row, pl.ds(off + 16, _L)], Rq[0], Rq[1])
                    return _merge_pair(
                        bufs[m, row, pl.ds(off + 32, _L)],
                        bufs[m, row, pl.ds(off + 48, _L)], q0, q1)

                return tuple(duo(row, Rq) for row, Rq in zip(rows, R))

            init = tuple((inf16, inf16) for _ in rows)
            R = lax.fori_loop(0, _D // 64, chunk_body, init)
            for Rq, row in zip(R, rows):
                finish(Rq, row)
            return carry

        lax.fori_loop(0, _G // 16, row_body, jnp.int32(0))

    sems = (sem_a, sem_b, sem_c)

    def sem_switch(m, fn):
        for s in range(3):
            @pl.when(m == s)
            def _(s=s):
                fn(sems[s])

    start(0, jnp.int32(0), sem_a)
    start(1, jnp.int32(1), sem_b)

    def group(g, carry):
        m = lax.rem(g, 3)

        @pl.when(g + 2 < _NG)
        def _():
            m2 = lax.rem(g + 2, 3)
            sem_switch(m2, lambda s: start(g + 2, m2, s))

        sem_switch(m, lambda s: drain(m, s))
        process(m, g)
        return carry

    lax.fori_loop(0, _NG, group, jnp.int32(0))

    pltpu.sync_copy(out_v, out.at[:, pl.ds(base, _BPW)])


@functools.partial(
    pl.kernel,
    mesh=plsc.VectorSubcoreMesh(core_axis_name="c", subcore_axis_name="s"),
    out_type=jax.ShapeDtypeStruct((_K, _B), jnp.float32),
    compiler_params=pltpu.CompilerParams(needs_layout_passes=False),
    scratch_types=[
        pltpu.VMEM((_BPW,), jnp.int32),
        pltpu.VMEM((3, _G, _D), jnp.float32),
        pltpu.VMEM((_K, _BPW), jnp.float32),
        pltpu.SemaphoreType.DMA,
        pltpu.SemaphoreType.DMA,
        pltpu.SemaphoreType.DMA,
    ],
)
def _bottom_k_sc(encoded, labels, out, lab_v, bufs, out_v, sem_a, sem_b, sem_c):
    _sc_body(encoded, labels, out, lab_v, bufs, out_v, sem_a, sem_b, sem_c)


def _softmax_cols(x):
    m = jnp.max(x, axis=0, keepdims=True)
    e = jnp.exp(x - m)
    return e / jnp.sum(e, axis=0, keepdims=True)


def _kl_body(rho_ref, rhohat_ref, out_ref):
    p = _softmax_cols(rho_ref[...])
    q = _softmax_cols(rhohat_ref[...])
    s1 = jnp.sum(p * jnp.log(p / q))
    s2 = jnp.sum((1.0 - p) * jnp.log((1.0 - p) / (1.0 - q)))
    out_ref[0, 0] = s1 + s2


def kernel(rho, encoded, labels, K):
    enc_t = jnp.swapaxes(encoded, 0, 1)
    table = enc_t.reshape(_C * _B, _D)
    labels32 = labels.astype(jnp.int32)
    rho_hat_t = _bottom_k_sc(table, labels32)
    rho_t = jnp.swapaxes(rho, 0, 1)
    loss = pl.pallas_call(
        _kl_body,
        out_shape=jax.ShapeDtypeStruct((1, 1), jnp.float32),
        out_specs=pl.BlockSpec(memory_space=pltpu.SMEM),
    )(rho_t, rho_hat_t)
    return loss[0, 0]


# speedup vs baseline: 1.1592x; 1.1592x over previous
import functools

import jax
import jax.numpy as jnp
from jax import lax
from jax.experimental import pallas as pl
from jax.experimental.pallas import tpu as pltpu
from jax.experimental.pallas import tpu_sc as plsc

_B = 4096
_C = 26
_D = 1024
_K = 32
_L = 16
_NC = 2
_NS = 16
_NW = _NC * _NS
_BPW = _B // _NW
_G = 16
_NG = _BPW // _G


def _rev(x):
    return lax.rev(x, (0,))


def _sort16(x):
    return plsc.sort_key_val(x, x)[0]


def _sort16d(x):
    return plsc.sort_key_val(x, x, descending=True)[0]


def _merge_pair(v0, v1, R0, R1):
    a = _sort16(v0)
    bd = _sort16d(v1)
    u0d = _sort16d(jnp.minimum(a, bd))
    u1d = _sort16d(jnp.maximum(a, bd))
    m0 = jnp.minimum(R0, u1d)
    m1 = jnp.minimum(R1, u0d)
    lo = jnp.minimum(m0, m1)
    hi = jnp.maximum(m0, m1)
    return _sort16(lo), _sort16(hi)


def _sc_body(encoded, labels, out, lab_v, bufs, out_v, sem_a, sem_b, sem_c):
    wid = lax.axis_index("s") * _NC + lax.axis_index("c")
    base = wid * _BPW

    pltpu.sync_copy(labels.at[pl.ds(base, _BPW)], lab_v)
    lane = lax.iota(jnp.int32, _L)
    lane_hi = lane + jnp.int32(_L)

    def start(g, m, sem):
        for j in range(_G // _L):
            chunk = lab_v[pl.ds(g * _G + j * _L, _L)]
            idx = chunk * jnp.int32(_B) + (base + g * _G + j * _L + lane)
            pltpu.async_copy(
                encoded.at[idx], bufs.at[m, pl.ds(j * _L, _L)], sem)

    def drain(m, sem):
        pltpu.make_async_copy(encoded.at[pl.ds(0, _G)], bufs.at[m], sem).wait()

    def process(m, g):
        inf16 = jnp.full((_L,), jnp.inf, jnp.float32)

        def finish(R, r):
            col = jnp.full((_L,), g * _G + r, jnp.int32)
            plsc.store_scatter(out_v, [lane, col], R[0])
            plsc.store_scatter(out_v, [lane_hi, col], R[1])

        def row_body(r, carry):
            rows = [8 * r + k for k in range(8)]

            def chunk_body(c, R):
                off = c * 64

                def duo(row, Rq):
                    q0, q1 = _merge_pair(
                        bufs[m, row, pl.ds(off, _L)],
                        bufs[m, row, pl.ds(off + 16, _L)], Rq[0], Rq[1])
                    return _merge_pair(
                        bufs[m, row, pl.ds(off + 32, _L)],
                        bufs[m, row, pl.ds(off + 48, _L)], q0, q1)

                return tuple(duo(row, Rq) for row, Rq in zip(rows, R))

            init = tuple((inf16, inf16) for _ in rows)
            R = lax.fori_loop(0, _D // 64, chunk_body, init)
            for Rq, row in zip(R, rows):
                finish(Rq, row)
            return carry

        lax.fori_loop(0, _G // 8, row_body, jnp.int32(0))

    sems = (sem_a, sem_b, sem_c)

    def sem_switch(m, fn):
        for s in range(3):
            @pl.when(m == s)
            def _(s=s):
                fn(sems[s])

    start(0, jnp.int32(0), sem_a)
    start(1, jnp.int32(1), sem_b)

    def group(g, carry):
        m = lax.rem(g, 3)

        @pl.when(g + 2 < _NG)
        def _():
            m2 = lax.rem(g + 2, 3)
            sem_switch(m2, lambda s: start(g + 2, m2, s))

        sem_switch(m, lambda s: drain(m, s))
        process(m, g)
        return carry

    lax.fori_loop(0, _NG, group, jnp.int32(0))

    pltpu.sync_copy(out_v, out.at[:, pl.ds(base, _BPW)])


@functools.partial(
    pl.kernel,
    mesh=plsc.VectorSubcoreMesh(core_axis_name="c", subcore_axis_name="s"),
    out_type=jax.ShapeDtypeStruct((_K, _B), jnp.float32),
    compiler_params=pltpu.CompilerParams(needs_layout_passes=False),
    scratch_types=[
        pltpu.VMEM((_BPW,), jnp.int32),
        pltpu.VMEM((3, _G, _D), jnp.float32),
        pltpu.VMEM((_K, _BPW), jnp.float32),
        pltpu.SemaphoreType.DMA,
        pltpu.SemaphoreType.DMA,
        pltpu.SemaphoreType.DMA,
    ],
)
def _bottom_k_sc(encoded, labels, out, lab_v, bufs, out_v, sem_a, sem_b, sem_c):
    _sc_body(encoded, labels, out, lab_v, bufs, out_v, sem_a, sem_b, sem_c)


def _softmax_cols(x):
    m = jnp.max(x, axis=0, keepdims=True)
    e = jnp.exp(x - m)
    return e / jnp.sum(e, axis=0, keepdims=True)


def _kl_body(rho_ref, rhohat_ref, out_ref):
    p = _softmax_cols(rho_ref[...])
    q = _softmax_cols(rhohat_ref[...])
    s1 = jnp.sum(p * jnp.log(p / q))
    s2 = jnp.sum((1.0 - p) * jnp.log((1.0 - p) / (1.0 - q)))
    out_ref[0, 0] = s1 + s2


def kernel(rho, encoded, labels, K):
    enc_t = jnp.swapaxes(encoded, 0, 1)
    table = enc_t.reshape(_C * _B, _D)
    labels32 = labels.astype(jnp.int32)
    rho_hat_t = _bottom_k_sc(table, labels32)
    rho_t = jnp.swapaxes(rho, 0, 1)
    loss = pl.pallas_call(
        _kl_body,
        out_shape=jax.ShapeDtypeStruct((1, 1), jnp.float32),
        out_specs=pl.BlockSpec(memory_space=pltpu.SMEM),
    )(rho_t, rho_hat_t)
    return loss[0, 0]
